# baseline (speedup 1.0000x reference)
import functools

import jax
import jax.numpy as jnp
from jax import lax
from jax.experimental import pallas as pl
from jax.experimental.pallas import tpu as pltpu
from jax.experimental.pallas import tpu_sc as plsc

_B, _C, _H, _W = 8, 128, 224, 224
_HW = _H * _W
_N = _B * _HW
_V = 1000
_VP = 1024
_NC, _NS = 2, 16
_NW = _NC * _NS
_CSPL = 2
_CW = _C // _CSPL
_NR = _NW // _CSPL
_PPW = _N // _NR
_K = 448
_T = _PPW // _K


def _transpose_table(tpad):
    def body(t_ref, o_ref):
        o_ref[...] = t_ref[...].T

    return pl.pallas_call(
        body, out_shape=jax.ShapeDtypeStruct((_C, _VP), jnp.float32)
    )(tpad)


def _sc_gather(table_t, idx):
    mesh = plsc.VectorSubcoreMesh(
        core_axis_name="c", subcore_axis_name="s",
        num_cores=_NC, num_subcores=_NS)

    @functools.partial(
        pl.kernel,
        out_type=jax.ShapeDtypeStruct((_N, _C), jnp.float32),
        mesh=mesh,
        compiler_params=pltpu.CompilerParams(
            needs_layout_passes=False, use_tc_tiling_on_sc=False),
        scratch_types=[
            pltpu.VMEM((_CW * _VP,), jnp.float32),
            pltpu.VMEM((_K,), jnp.int32),
            pltpu.VMEM((_K,), jnp.int32),
            pltpu.VMEM((_K, _CW), jnp.float32),
            pltpu.VMEM((_K, _CW), jnp.float32),
            pltpu.SemaphoreType.DMA,
            pltpu.SemaphoreType.DMA,
            pltpu.SemaphoreType.DMA,
            pltpu.SemaphoreType.DMA,
            pltpu.SemaphoreType.DMA,
        ],
    )
    def k(tt_hbm, idx_hbm, out_hbm, lut_v, idx0, idx1, st0, st1,
          sem_lut, sem_i0, sem_i1, sem_o0, sem_o1):
        wid = lax.axis_index("s") * _NC + lax.axis_index("c")
        rng = wid // _CSPL
        half = wid % _CSPL
        base = rng * _PPW
        c0 = half * _CW
        idx_bufs = (idx0, idx1)
        stages = (st0, st1)
        sem_is = (sem_i0, sem_i1)
        sem_os = (sem_o0, sem_o1)
        lanes = jnp.arange(16, dtype=jnp.int32)
        colvecs = [jnp.full((16,), c, jnp.int32) for c in range(_CW)]

        lut_copy = pltpu.async_copy(
            tt_hbm.at[pl.ds(c0 * _VP, _CW * _VP)], lut_v, sem_lut)
        pltpu.async_copy(idx_hbm.at[pl.ds(base, _K)], idx0, sem_i0)
        pltpu.async_copy(idx_hbm.at[pl.ds(base + _K, _K)], idx1, sem_i1)
        lut_copy.wait()

        def chunk_body(t, _):
            for s in (0, 1):
                te = t + s
                idx_v, stage_v = idx_bufs[s], stages[s]
                r0 = base + te * _K
                pltpu.make_async_copy(
                    idx_hbm.at[pl.ds(0, _K)], idx_v, sem_is[s]).wait()
                @pl.when(te >= 2)
                def _drain_out():
                    pltpu.make_async_copy(
                        stage_v,
                        out_hbm.at[pl.ds(0, _K), pl.ds(0, _CW)],
                        sem_os[s]).wait()

                @plsc.parallel_loop(0, _K // 16, unroll=2)
                def _pos_body(i):
                    iv16 = idx_v[pl.ds(i * 16, 16)]
                    rows = lanes + i * 16
                    for c in range(_CW):
                        vals = plsc.load_gather(lut_v, [iv16 + c * _VP])
                        plsc.store_scatter(stage_v, [rows, colvecs[c]], vals)

                pltpu.async_copy(
                    stage_v, out_hbm.at[pl.ds(r0, _K), pl.ds(c0, _CW)],
                    sem_os[s])
                @pl.when(te + 2 < _T)
                def _prefetch():
                    pltpu.async_copy(
                        idx_hbm.at[pl.ds(base + (te + 2) * _K, _K)], idx_v,
                        sem_is[s])
            return 0

        lax.fori_loop(0, _T // 2, lambda u, c: chunk_body(u * 2, c), 0)
        for s in (0, 1):
            pltpu.make_async_copy(
                stages[s], out_hbm.at[pl.ds(0, _K), pl.ds(0, _CW)],
                sem_os[s]).wait()

    return k(table_t, idx)


def kernel(x, table):
    idx = x.reshape(_N)
    tpad = jnp.zeros((_VP, _C), jnp.float32).at[:_V].set(table)
    table_t = _transpose_table(tpad).reshape(_C * _VP)
    emb = _sc_gather(table_t, idx)
    emb = emb.reshape(_B, _H, _W, _C)
    return jnp.transpose(emb, (0, 3, 1, 2))


# speedup vs baseline: 3.7026x; 3.7026x over previous
import functools

import jax
import jax.numpy as jnp
from jax import lax
from jax.experimental import pallas as pl
from jax.experimental.pallas import tpu as pltpu
from jax.experimental.pallas import tpu_sc as plsc

_B, _C, _H, _W = 8, 128, 224, 224
_HW = _H * _W
_N = _B * _HW
_NC, _NS = 2, 16
_NW = _NC * _NS
_RPW = _N // _NW
_GS = 112
_K = 4 * _GS
_T = _RPW // _K


def _sc_gather(table, idx):
    mesh = plsc.VectorSubcoreMesh(
        core_axis_name="c", subcore_axis_name="s",
        num_cores=_NC, num_subcores=_NS)

    @functools.partial(
        pl.kernel,
        out_type=jax.ShapeDtypeStruct((_N, _C), jnp.float32),
        mesh=mesh,
        compiler_params=pltpu.CompilerParams(needs_layout_passes=False),
        scratch_types=[
            pltpu.VMEM((_K,), jnp.int32),
            pltpu.VMEM((_K,), jnp.int32),
            pltpu.VMEM((_K, _C), jnp.float32),
            pltpu.VMEM((_K, _C), jnp.float32),
            pltpu.SemaphoreType.DMA,
            pltpu.SemaphoreType.DMA,
            pltpu.SemaphoreType.DMA,
            pltpu.SemaphoreType.DMA,
            pltpu.SemaphoreType.DMA,
            pltpu.SemaphoreType.DMA,
        ],
    )
    def k(tab_hbm, idx_hbm, out_hbm, idx0, idx1, rows0, rows1,
          sem_i0, sem_i1, sem_g0, sem_g1, sem_o0, sem_o1):
        wid = lax.axis_index("s") * _NC + lax.axis_index("c")
        base = wid * _RPW
        idx_bufs = (idx0, idx1)
        row_bufs = (rows0, rows1)
        sem_is = (sem_i0, sem_i1)
        sem_gs = (sem_g0, sem_g1)
        sem_os = (sem_o0, sem_o1)

        pltpu.async_copy(idx_hbm.at[pl.ds(base, _K)], idx0, sem_i0)
        pltpu.async_copy(idx_hbm.at[pl.ds(base + _K, _K)], idx1, sem_i1)

        def chunk_body(t, _):
            for s in (0, 1):
                te = t + s
                idx_v, rows_v = idx_bufs[s], row_bufs[s]
                r0 = base + te * _K
                pltpu.make_async_copy(
                    idx_hbm.at[pl.ds(0, _K)], idx_v, sem_is[s]).wait()
                @pl.when(te >= 2)
                def _drain_out():
                    pltpu.make_async_copy(
                        rows_v, out_hbm.at[pl.ds(0, _K)], sem_os[s]).wait()

                for g in range(_K // _GS):
                    pltpu.async_copy(
                        tab_hbm.at[idx_v.at[pl.ds(g * _GS, _GS)]],
                        rows_v.at[pl.ds(g * _GS, _GS)], sem_gs[s])
                for g in range(_K // _GS):
                    pltpu.make_async_copy(
                        tab_hbm.at[idx_v.at[pl.ds(0, _GS)]],
                        rows_v.at[pl.ds(0, _GS)], sem_gs[s]).wait()
                pltpu.async_copy(rows_v, out_hbm.at[pl.ds(r0, _K)], sem_os[s])
                @pl.when(te + 2 < _T)
                def _prefetch():
                    pltpu.async_copy(
                        idx_hbm.at[pl.ds(base + (te + 2) * _K, _K)], idx_v,
                        sem_is[s])
            return 0

        lax.fori_loop(0, _T // 2, lambda u, c: chunk_body(u * 2, c), 0)
        for s in (0, 1):
            pltpu.make_async_copy(
                row_bufs[s], out_hbm.at[pl.ds(0, _K)], sem_os[s]).wait()

    return k(table, idx)


def kernel(x, table):
    idx = x.reshape(_N)
    emb = _sc_gather(table, idx)
    emb = emb.reshape(_B, _H, _W, _C)
    return jnp.transpose(emb, (0, 3, 1, 2))


# speedup vs baseline: 3.7200x; 1.0047x over previous
import functools

import jax
import jax.numpy as jnp
from jax import lax
from jax.experimental import pallas as pl
from jax.experimental.pallas import tpu as pltpu
from jax.experimental.pallas import tpu_sc as plsc

_B, _C, _H, _W = 8, 128, 224, 224
_HW = _H * _W
_N = _B * _HW
_NC, _NS = 2, 16
_NW = _NC * _NS
_RPW = _N // _NW
_GS = 112
_K = 2 * _GS
_T = _RPW // _K
_D = 4


def _sc_gather(table, idx):
    mesh = plsc.VectorSubcoreMesh(
        core_axis_name="c", subcore_axis_name="s",
        num_cores=_NC, num_subcores=_NS)

    @functools.partial(
        pl.kernel,
        out_type=jax.ShapeDtypeStruct((_N, _C), jnp.float32),
        mesh=mesh,
        compiler_params=pltpu.CompilerParams(needs_layout_passes=False),
        scratch_types=(
            [pltpu.VMEM((_K,), jnp.int32) for _ in range(_D)]
            + [pltpu.VMEM((_K, _C), jnp.float32) for _ in range(_D)]
            + [pltpu.SemaphoreType.DMA for _ in range(3 * _D)]
        ),
    )
    def k(tab_hbm, idx_hbm, out_hbm, *bufs):
        idx_bufs = bufs[:_D]
        row_bufs = bufs[_D:2 * _D]
        sem_is = bufs[2 * _D:3 * _D]
        sem_gs = bufs[3 * _D:4 * _D]
        sem_os = bufs[4 * _D:5 * _D]
        wid = lax.axis_index("s") * _NC + lax.axis_index("c")
        base = wid * _RPW

        for s in range(_D):
            pltpu.async_copy(idx_hbm.at[pl.ds(base + s * _K, _K)],
                             idx_bufs[s], sem_is[s])

        def chunk_body(t, _):
            for s in range(_D):
                te = t + s
                idx_v, rows_v = idx_bufs[s], row_bufs[s]
                r0 = base + te * _K
                pltpu.make_async_copy(
                    idx_hbm.at[pl.ds(0, _K)], idx_v, sem_is[s]).wait()
                @pl.when(te >= _D)
                def _drain_out():
                    pltpu.make_async_copy(
                        rows_v, out_hbm.at[pl.ds(0, _K)], sem_os[s]).wait()

                for g in range(_K // _GS):
                    pltpu.async_copy(
                        tab_hbm.at[idx_v.at[pl.ds(g * _GS, _GS)]],
                        rows_v.at[pl.ds(g * _GS, _GS)], sem_gs[s])
                for g in range(_K // _GS):
                    pltpu.make_async_copy(
                        tab_hbm.at[idx_v.at[pl.ds(0, _GS)]],
                        rows_v.at[pl.ds(0, _GS)], sem_gs[s]).wait()
                pltpu.async_copy(rows_v, out_hbm.at[pl.ds(r0, _K)], sem_os[s])
                @pl.when(te + _D < _T)
                def _prefetch():
                    pltpu.async_copy(
                        idx_hbm.at[pl.ds(base + (te + _D) * _K, _K)], idx_v,
                        sem_is[s])
            return 0

        lax.fori_loop(0, _T // _D, lambda u, c: chunk_body(u * _D, c), 0)
        for s in range(_D):
            pltpu.make_async_copy(
                row_bufs[s], out_hbm.at[pl.ds(0, _K)], sem_os[s]).wait()

    return k(table, idx)


def kernel(x, table):
    idx = x.reshape(_N)
    emb = _sc_gather(table, idx)
    emb = emb.reshape(_B, _H, _W, _C)
    return jnp.transpose(emb, (0, 3, 1, 2))
